# Initial kernel scaffold; baseline (speedup 1.0000x reference)
import jax
import jax.numpy as jnp
from jax.experimental import pallas as pl


def kernel(batch_sequence_tensors, regular_tokens_mask):
    raise NotImplementedError("write your pallas kernel here")


# speedup vs baseline: 16.8512x; 16.8512x over previous
import functools

import jax
import jax.numpy as jnp
from jax.experimental import pallas as pl
from jax.experimental.pallas import tpu as pltpu

_LIMIT = 8
_THRESHOLD = 0.9
_EPS = 1e-5


def _shift(a, c):
    if c == 0:
        return a
    return pltpu.roll(a, a.shape[1] - c, axis=1)


def _band_kernel(xt_ref, out_ref):
    xt = xt_ref[0]
    _, s_len = xt.shape
    rows = []
    for d in range(_LIMIT):
        sh = xt if d == 0 else pltpu.roll(xt, s_len - d, axis=1)
        r = jnp.sum(xt * sh, axis=0, keepdims=True)
        if d > 0:
            lane = jax.lax.broadcasted_iota(jnp.int32, (1, s_len), 1)
            r = jnp.where(lane < s_len - d, r, 0.0)
        rows.append(r)
    out_ref[0] = jnp.concatenate(rows, axis=0)


def _window_kernel(bands_ref, rm_ref, worst_ref, incl_ref):
    a = [bands_ref[d] for d in range(_LIMIT)]
    rmf = rm_ref[...]

    n = a[0]
    gn = jnp.maximum(jnp.sqrt(n), _EPS)
    gns = [_shift(gn, j) for j in range(_LIMIT)]

    rows = [n] + [None] * (_LIMIT - 1)
    swin = n
    sims1 = n / (jnp.maximum(jnp.sqrt(n), _EPS) * gns[0])
    worst_ref[0] = sims1
    incl_ref[0] = jnp.ones_like(rmf)

    regw = rmf
    for L in range(2, _LIMIT + 1):
        for j in range(L - 1):
            rows[j] = rows[j] + _shift(a[L - 1 - j], j)
        new = a[L - 1]
        for k in range(1, L):
            new = new + _shift(a[L - 1 - k], k)
        swin = swin + 2.0 * new - _shift(n, L - 1)
        rows[L - 1] = new
        regw = regw * _shift(rmf, L - 1)

        cn_inv = 1.0 / jnp.maximum(jnp.sqrt(jnp.maximum(swin, 0.0)),
                                   L * _EPS)
        worst = None
        for j in range(L):
            s_j = rows[j] * cn_inv / gns[j]
            worst = s_j if worst is None else jnp.minimum(worst, s_j)
        worst_ref[L - 1] = worst
        incl_ref[L - 1] = jnp.where((worst >= _THRESHOLD) & (regw == 1), 1, 0)


@functools.partial(jax.jit, static_argnums=())
def kernel(batch_sequence_tensors, regular_tokens_mask):
    x = batch_sequence_tensors
    rm = regular_tokens_mask.astype(jnp.int32)
    b, s_len, d_len = x.shape

    xt = jnp.swapaxes(x, 1, 2)
    bands = pl.pallas_call(
        _band_kernel,
        grid=(b,),
        in_specs=[pl.BlockSpec((1, d_len, s_len), lambda i: (i, 0, 0))],
        out_specs=pl.BlockSpec((1, _LIMIT, s_len), lambda i: (i, 0, 0)),
        out_shape=jax.ShapeDtypeStruct((b, _LIMIT, s_len), jnp.float32),
    )(xt)

    bands_t = jnp.swapaxes(bands, 0, 1)
    worst8, incl8 = pl.pallas_call(
        _window_kernel,
        out_shape=(
            jax.ShapeDtypeStruct((_LIMIT, b, s_len), jnp.float32),
            jax.ShapeDtypeStruct((_LIMIT, b, s_len), jnp.int32),
        ),
    )(bands_t, rm)

    worst_all = jnp.concatenate(
        [worst8[L - 1, :, : s_len - L + 1] for L in range(1, _LIMIT + 1)],
        axis=1)
    include = jnp.concatenate(
        [incl8[L - 1, :, : s_len - L + 1] != 0 for L in range(1, _LIMIT + 1)],
        axis=1)
    return worst_all, include


# speedup vs baseline: 55.7086x; 3.3059x over previous
import jax
import jax.numpy as jnp
from jax.experimental import pallas as pl
from jax.experimental.pallas import tpu as pltpu

_LIMIT = 8
_THRESHOLD = 0.9
_EPS = 1e-5


def _shift(a, c):
    if c == 0:
        return a
    return pltpu.roll(a, a.shape[1] - c, axis=1)


def _band_kernel(x_ref, out_ref):
    x = x_ref[0]
    s_len, _ = x.shape
    cols = []
    for d in range(_LIMIT):
        sh = x if d == 0 else pltpu.roll(x, s_len - d, axis=0)
        c = jnp.sum(x * sh, axis=1, keepdims=True)
        if d > 0:
            sub = jax.lax.broadcasted_iota(jnp.int32, (s_len, 1), 0)
            c = jnp.where(sub < s_len - d, c, 0.0)
        cols.append(c)
    out_ref[0] = jnp.concatenate(cols, axis=1)


def _window_kernel(bands_ref, rm_ref, worst_ref, incl_ref):
    a = [bands_ref[d] for d in range(_LIMIT)]
    rmf = rm_ref[...]
    s_len = rmf.shape[1]

    n = a[0]
    gns = [_shift(jnp.maximum(jnp.sqrt(n), _EPS), j) for j in range(_LIMIT)]

    rows = [n] + [None] * (_LIMIT - 1)
    swin = n
    off = 0
    worst_ref[:, :s_len] = n / (jnp.maximum(jnp.sqrt(n), _EPS) * gns[0])
    incl_ref[:, :s_len] = jnp.ones_like(rmf)
    off += s_len

    regw = rmf
    for L in range(2, _LIMIT + 1):
        for j in range(L - 1):
            rows[j] = rows[j] + _shift(a[L - 1 - j], j)
        new = a[L - 1]
        for k in range(1, L):
            new = new + _shift(a[L - 1 - k], k)
        swin = swin + 2.0 * new - _shift(n, L - 1)
        rows[L - 1] = new
        regw = regw * _shift(rmf, L - 1)

        cn_inv = 1.0 / jnp.maximum(jnp.sqrt(jnp.maximum(swin, 0.0)),
                                   L * _EPS)
        worst = None
        for j in range(L):
            s_j = rows[j] * cn_inv / gns[j]
            worst = s_j if worst is None else jnp.minimum(worst, s_j)
        w = s_len - L + 1
        worst_ref[:, off:off + w] = worst[:, :w]
        incl_ref[:, off:off + w] = jnp.where(
            (worst >= _THRESHOLD) & (regw == 1), 1, 0)[:, :w]
        off += w


def kernel(batch_sequence_tensors, regular_tokens_mask):
    x = batch_sequence_tensors
    rm = regular_tokens_mask.astype(jnp.int32)
    b, s_len, d_len = x.shape
    n_out = _LIMIT * s_len - (_LIMIT * (_LIMIT - 1)) // 2

    bands = pl.pallas_call(
        _band_kernel,
        grid=(b,),
        in_specs=[pl.BlockSpec((1, s_len, d_len), lambda i: (i, 0, 0))],
        out_specs=pl.BlockSpec((1, s_len, _LIMIT), lambda i: (i, 0, 0)),
        out_shape=jax.ShapeDtypeStruct((b, s_len, _LIMIT), jnp.float32),
    )(x)

    bands_t = jnp.transpose(bands, (2, 0, 1))
    worst_all, incl = pl.pallas_call(
        _window_kernel,
        out_shape=(
            jax.ShapeDtypeStruct((b, n_out), jnp.float32),
            jax.ShapeDtypeStruct((b, n_out), jnp.int32),
        ),
    )(bands_t, rm)

    return worst_all, incl != 0


# speedup vs baseline: 68.5067x; 1.2297x over previous
import jax
import jax.numpy as jnp
from jax.experimental import pallas as pl
from jax.experimental.pallas import tpu as pltpu

_LIMIT = 8
_THRESHOLD = 0.9
_EPS = 1e-5


def _shift(a, c):
    if c == 0:
        return a
    return pltpu.roll(a, a.shape[1] - c, axis=1)


def _fused_kernel(x_ref, rm_ref, worst_ref, incl_ref):
    b, s_len, _ = x_ref.shape

    cols = [None] * (_LIMIT * b)
    for bi in range(b):
        x = x_ref[bi]
        for d in range(_LIMIT):
            sh = x if d == 0 else pltpu.roll(x, s_len - d, axis=0)
            c = jnp.sum(x * sh, axis=1, keepdims=True)
            if d > 0:
                sub = jax.lax.broadcasted_iota(jnp.int32, (s_len, 1), 0)
                c = jnp.where(sub < s_len - d, c, 0.0)
            cols[d * b + bi] = c
    m = jnp.concatenate(cols, axis=1)
    mt = m.T
    a = [jax.lax.slice_in_dim(mt, d * b, (d + 1) * b, axis=0)
         for d in range(_LIMIT)]

    rmf = rm_ref[...]
    n = a[0]
    gns = [_shift(jnp.maximum(jnp.sqrt(n), _EPS), j) for j in range(_LIMIT)]

    rows = [n] + [None] * (_LIMIT - 1)
    swin = n
    off = 0
    worst_ref[:, :s_len] = n / (jnp.maximum(jnp.sqrt(n), _EPS) * gns[0])
    incl_ref[:, :s_len] = jnp.ones_like(rmf)
    off += s_len

    regw = rmf
    for L in range(2, _LIMIT + 1):
        for j in range(L - 1):
            rows[j] = rows[j] + _shift(a[L - 1 - j], j)
        new = a[L - 1]
        for k in range(1, L):
            new = new + _shift(a[L - 1 - k], k)
        swin = swin + 2.0 * new - _shift(n, L - 1)
        rows[L - 1] = new
        regw = regw * _shift(rmf, L - 1)

        cn_inv = 1.0 / jnp.maximum(jnp.sqrt(jnp.maximum(swin, 0.0)),
                                   L * _EPS)
        worst = None
        for j in range(L):
            s_j = rows[j] * cn_inv / gns[j]
            worst = s_j if worst is None else jnp.minimum(worst, s_j)
        w = s_len - L + 1
        worst_ref[:, off:off + w] = worst[:, :w]
        incl_ref[:, off:off + w] = jnp.where(
            (worst >= _THRESHOLD) & (regw == 1), 1, 0)[:, :w]
        off += w


def kernel(batch_sequence_tensors, regular_tokens_mask):
    x = batch_sequence_tensors
    rm = regular_tokens_mask.astype(jnp.int32)
    b, s_len, _ = x.shape
    n_out = _LIMIT * s_len - (_LIMIT * (_LIMIT - 1)) // 2

    worst_all, incl = pl.pallas_call(
        _fused_kernel,
        out_shape=(
            jax.ShapeDtypeStruct((b, n_out), jnp.float32),
            jax.ShapeDtypeStruct((b, n_out), jnp.int32),
        ),
    )(x, rm)

    return worst_all, incl != 0
